# baseline (speedup 1.0000x reference)
import functools

import jax
import jax.numpy as jnp
from jax import lax
from jax.experimental import pallas as pl
from jax.experimental.pallas import tpu as pltpu
from jax.experimental.pallas import tpu_sc as plsc

BB = 4
NN = 50000
KK = 128
NBR = 3
EPS_NORM = 1e-12
EPS_BN = 1e-5


_TA = 1280


def _prep_normals_body(nrm_ref, xnp_ref):
    x = nrm_ref[0]
    s = jnp.sum(x * x, axis=0, keepdims=True)
    inv = 1.0 / jnp.maximum(jnp.sqrt(s), EPS_NORM)
    xn = x * inv
    xpad = jnp.concatenate([xn, jnp.zeros((13, _TA), jnp.float32)], axis=0)
    xnp_ref[0] = xpad.T


def _prep_dirs_body(d_ref, dnp_ref):
    d = d_ref[...]
    s = jnp.sum(d * d, axis=0, keepdims=True)
    inv = 1.0 / jnp.maximum(jnp.sqrt(s), EPS_NORM)
    dn = d * inv
    dnp_ref[...] = jnp.concatenate([dn, jnp.zeros((13, KK), jnp.float32)], axis=0)


_R = 400
_G = 80
_NG = (_R * NBR) // _G
_NCHUNK = (BB * NN) // _R
_CPB = NN // _R
_NW = 32


def _gather_kernel(ring_hbm, xnp_hbm, out_hbm, idx_v, rows_v, sem):
    cid = lax.axis_index("c")
    sid = lax.axis_index("s")
    wid = sid * 2 + cid

    def chunk_body(t, _):
        c = wid + t * _NW
        base = c * (_R * NBR)
        boff = (c // _CPB) * NN

        pltpu.sync_copy(ring_hbm.at[c], idx_v)

        def add_body(i, _):
            g = i // (_G // 16)
            j = i % (_G // 16)
            sl = pl.ds(j * 16, 16)
            idx_v[g, sl] = idx_v[g, sl] + boff
            return 0

        lax.fori_loop(0, (_R * NBR) // 16, add_body, 0)

        descs = []
        for g in range(_NG):
            descs.append(
                pltpu.async_copy(
                    xnp_hbm.at[idx_v.at[g]],
                    rows_v.at[pl.ds(g * _G, _G)],
                    sem,
                )
            )
        for d in descs:
            d.wait()
        pltpu.sync_copy(rows_v, out_hbm.at[pl.ds(base, _R * NBR)])
        return 0

    nchunks = (_NCHUNK - wid + _NW - 1) // _NW
    lax.fori_loop(0, nchunks, chunk_body, 0)


_TD = 2000
_TE = 1280


def _feat_block(g, xs, dnp):
    pg = lax.dot_general(g, dnp, (((1,), (0,)), ((), ())),
                         preferred_element_type=jnp.float32)
    ps = lax.dot_general(xs, dnp, (((1,), (0,)), ((), ())),
                         preferred_element_type=jnp.float32)
    t = xs.shape[0]
    m3 = jnp.max(pg.reshape(t, NBR, KK), axis=1)
    return jnp.maximum(m3, ps)


def _stats_body(g_ref, x_ref, dnp_ref, sum_ref, sq_ref):
    b = pl.program_id(0)
    i = pl.program_id(1)
    feat = _feat_block(g_ref[0], x_ref[0], dnp_ref[...])
    s = jnp.sum(feat, axis=0, keepdims=True)
    q = jnp.sum(feat * feat, axis=0, keepdims=True)

    @pl.when(jnp.logical_and(b == 0, i == 0))
    def _():
        sum_ref[...] = jnp.zeros_like(sum_ref)
        sq_ref[...] = jnp.zeros_like(sq_ref)

    sum_ref[0:1, :] += s
    sq_ref[0:1, :] += q


def _apply_body(g_ref, x_ref, dnp_ref, sum_ref, sq_ref, gm_ref, bt_ref, o_ref):
    feat = _feat_block(g_ref[0], x_ref[0], dnp_ref[...])
    cnt = float(BB * NN)
    mean = sum_ref[0:1, :] / cnt
    var = sq_ref[0:1, :] / cnt - mean * mean
    rstd = lax.rsqrt(var + EPS_BN)
    scale = gm_ref[...] * rstd
    shift = bt_ref[...] - mean * scale
    y = jnp.maximum(feat * scale + shift, 0.0)
    o_ref[0] = y.T


def kernel(normals, ring_n, directions, gamma, beta):
    xnp = pl.pallas_call(
        _prep_normals_body,
        grid=(BB, pl.cdiv(NN, _TA)),
        in_specs=[pl.BlockSpec((1, 3, _TA), lambda b, i: (b, 0, i))],
        out_specs=pl.BlockSpec((1, _TA, 16), lambda b, i: (b, i, 0)),
        out_shape=jax.ShapeDtypeStruct((BB, NN, 16), jnp.float32),
    )(normals)

    dnp = pl.pallas_call(
        _prep_dirs_body,
        out_shape=jax.ShapeDtypeStruct((16, KK), jnp.float32),
    )(directions)

    ring2 = ring_n.reshape(_NCHUNK, _NG, _G).astype(jnp.int32)
    xnp_flat = xnp.reshape(BB * NN, 16)

    mesh = plsc.VectorSubcoreMesh(core_axis_name="c", subcore_axis_name="s")
    gathered = functools.partial(
        pl.kernel,
        mesh=mesh,
        compiler_params=pltpu.CompilerParams(use_tc_tiling_on_sc=False),
        out_type=jax.ShapeDtypeStruct((BB * NN * NBR, 16), jnp.float32),
        scratch_types=[
            pltpu.VMEM((_NG, _G), jnp.int32),
            pltpu.VMEM((_R * NBR, 16), jnp.float32),
            pltpu.SemaphoreType.DMA,
        ],
    )(_gather_kernel)(ring2, xnp_flat)

    g3 = gathered.reshape(BB, NN * NBR, 16)

    sums, sqs = pl.pallas_call(
        _stats_body,
        grid=(BB, NN // _TD),
        in_specs=[
            pl.BlockSpec((1, NBR * _TD, 16), lambda b, i: (b, i, 0)),
            pl.BlockSpec((1, _TD, 16), lambda b, i: (b, i, 0)),
            pl.BlockSpec((16, KK), lambda b, i: (0, 0)),
        ],
        out_specs=[
            pl.BlockSpec((8, KK), lambda b, i: (0, 0)),
            pl.BlockSpec((8, KK), lambda b, i: (0, 0)),
        ],
        out_shape=[
            jax.ShapeDtypeStruct((8, KK), jnp.float32),
            jax.ShapeDtypeStruct((8, KK), jnp.float32),
        ],
    )(g3, xnp, dnp)

    gm = gamma.reshape(1, KK)
    bt = beta.reshape(1, KK)

    out = pl.pallas_call(
        _apply_body,
        grid=(BB, pl.cdiv(NN, _TE)),
        in_specs=[
            pl.BlockSpec((1, NBR * _TE, 16), lambda b, i: (b, i, 0)),
            pl.BlockSpec((1, _TE, 16), lambda b, i: (b, i, 0)),
            pl.BlockSpec((16, KK), lambda b, i: (0, 0)),
            pl.BlockSpec((8, KK), lambda b, i: (0, 0)),
            pl.BlockSpec((8, KK), lambda b, i: (0, 0)),
            pl.BlockSpec((1, KK), lambda b, i: (0, 0)),
            pl.BlockSpec((1, KK), lambda b, i: (0, 0)),
        ],
        out_specs=pl.BlockSpec((1, KK, _TE), lambda b, i: (b, 0, i)),
        out_shape=jax.ShapeDtypeStruct((BB, KK, NN), jnp.float32),
    )(g3, xnp, dnp, sums, sqs, gm, bt)

    return out


# speedup vs baseline: 2.1142x; 2.1142x over previous
import functools

import jax
import jax.numpy as jnp
from jax import lax
from jax.experimental import pallas as pl
from jax.experimental.pallas import tpu as pltpu
from jax.experimental.pallas import tpu_sc as plsc

BB = 4
NN = 50000
KK = 128
NBR = 3
EPS_NORM = 1e-12
EPS_BN = 1e-5


_TA = 2560


def _prep_normals_body(nrm_ref, xnp_ref):
    x = nrm_ref[0]
    s = jnp.sum(x * x, axis=0, keepdims=True)
    inv = 1.0 / jnp.maximum(jnp.sqrt(s), EPS_NORM)
    xn = x * inv
    xpad = jnp.concatenate([xn, jnp.zeros((13, _TA), jnp.float32)], axis=0)
    xnp_ref[0] = xpad.T


def _prep_dirs_body(d_ref, dnp_ref):
    d = d_ref[...]
    s = jnp.sum(d * d, axis=0, keepdims=True)
    inv = 1.0 / jnp.maximum(jnp.sqrt(s), EPS_NORM)
    dn = d * inv
    dnp_ref[...] = jnp.concatenate([dn, jnp.zeros((13, KK), jnp.float32)], axis=0)


_R = 400
_G = 80
_NG = (_R * NBR) // _G
_NCHUNK = (BB * NN) // _R
_CPB = NN // _R
_NW = 32
_GPP = _R // _G


def _gather_kernel(ring_hbm, xnp_hbm, out_hbm, raw_v, idx_v, rows_v, sem):
    cid = lax.axis_index("c")
    sid = lax.axis_index("s")
    wid = sid * 2 + cid

    lane = lax.iota(jnp.int32, 16)
    lane3 = lane * NBR

    def chunk_body(t, _):
        c = wid + t * _NW
        boff = (c // _CPB) * NN

        pltpu.sync_copy(ring_hbm.at[c], raw_v)

        for v in range((_R * NBR) // 16):
            p0 = v * 16
            j = p0 // _R
            r0 = p0 % _R
            src = plsc.load_gather(raw_v, [lane3 + (r0 * NBR + j)])
            idx_v[p0 // _G, pl.ds(p0 % _G, 16)] = src + boff

        descs = []
        for g in range(_NG):
            descs.append(
                pltpu.async_copy(
                    xnp_hbm.at[idx_v.at[g]],
                    rows_v.at[pl.ds(g * _G, _G)],
                    sem,
                )
            )
        for d in descs:
            d.wait()

        for j in range(NBR):
            pltpu.sync_copy(
                rows_v.at[pl.ds(j * _R, _R)],
                out_hbm.at[pl.ds(j * (BB * NN) + c * _R, _R)],
            )
        return 0

    nchunks = (_NCHUNK - wid + _NW - 1) // _NW
    lax.fori_loop(0, nchunks, chunk_body, 0)


_TD = 5000
_TE = 2560


def _feat_block(g3, xs, dnp):
    def dot16(a):
        return lax.dot_general(a, dnp, (((1,), (0,)), ((), ())),
                               preferred_element_type=jnp.float32)

    m01 = jnp.maximum(dot16(g3[0, 0]), dot16(g3[1, 0]))
    m2s = jnp.maximum(dot16(g3[2, 0]), dot16(xs))
    return jnp.maximum(m01, m2s)


def _stats_body(g_ref, x_ref, dnp_ref, sum_ref, sq_ref):
    b = pl.program_id(0)
    i = pl.program_id(1)
    feat = _feat_block(g_ref[...], x_ref[0], dnp_ref[...])
    s = jnp.sum(feat, axis=0, keepdims=True)
    q = jnp.sum(feat * feat, axis=0, keepdims=True)

    @pl.when(jnp.logical_and(b == 0, i == 0))
    def _():
        sum_ref[...] = jnp.zeros_like(sum_ref)
        sq_ref[...] = jnp.zeros_like(sq_ref)

    sum_ref[0:1, :] += s
    sq_ref[0:1, :] += q


def _apply_body(g_ref, x_ref, dnp_ref, sum_ref, sq_ref, gm_ref, bt_ref, o_ref):
    feat = _feat_block(g_ref[...], x_ref[0], dnp_ref[...])
    cnt = float(BB * NN)
    mean = sum_ref[0:1, :] / cnt
    var = sq_ref[0:1, :] / cnt - mean * mean
    rstd = lax.rsqrt(var + EPS_BN)
    scale = gm_ref[...] * rstd
    shift = bt_ref[...] - mean * scale
    y = jnp.maximum(feat * scale + shift, 0.0)
    o_ref[0] = y.T


def kernel(normals, ring_n, directions, gamma, beta):
    xnp = pl.pallas_call(
        _prep_normals_body,
        grid=(BB, pl.cdiv(NN, _TA)),
        in_specs=[pl.BlockSpec((1, 3, _TA), lambda b, i: (b, 0, i))],
        out_specs=pl.BlockSpec((1, _TA, 16), lambda b, i: (b, i, 0)),
        out_shape=jax.ShapeDtypeStruct((BB, NN, 16), jnp.float32),
    )(normals)

    dnp = pl.pallas_call(
        _prep_dirs_body,
        out_shape=jax.ShapeDtypeStruct((16, KK), jnp.float32),
    )(directions)

    ring2 = ring_n.reshape(_NCHUNK, _R * NBR).astype(jnp.int32)
    xnp_flat = xnp.reshape(BB * NN, 16)

    mesh = plsc.VectorSubcoreMesh(core_axis_name="c", subcore_axis_name="s")
    gathered = functools.partial(
        pl.kernel,
        mesh=mesh,
        compiler_params=pltpu.CompilerParams(
            use_tc_tiling_on_sc=False, needs_layout_passes=False),
        out_type=jax.ShapeDtypeStruct((NBR * BB * NN, 16), jnp.float32),
        scratch_types=[
            pltpu.VMEM((_R * NBR,), jnp.int32),
            pltpu.VMEM((_NG, _G), jnp.int32),
            pltpu.VMEM((_R * NBR, 16), jnp.float32),
            pltpu.SemaphoreType.DMA,
        ],
    )(_gather_kernel)(ring2, xnp_flat)

    g4 = gathered.reshape(NBR, BB, NN, 16)

    sums, sqs = pl.pallas_call(
        _stats_body,
        grid=(BB, NN // _TD),
        in_specs=[
            pl.BlockSpec((NBR, 1, _TD, 16), lambda b, i: (0, b, i, 0)),
            pl.BlockSpec((1, _TD, 16), lambda b, i: (b, i, 0)),
            pl.BlockSpec((16, KK), lambda b, i: (0, 0)),
        ],
        out_specs=[
            pl.BlockSpec((8, KK), lambda b, i: (0, 0)),
            pl.BlockSpec((8, KK), lambda b, i: (0, 0)),
        ],
        out_shape=[
            jax.ShapeDtypeStruct((8, KK), jnp.float32),
            jax.ShapeDtypeStruct((8, KK), jnp.float32),
        ],
    )(g4, xnp, dnp)

    gm = gamma.reshape(1, KK)
    bt = beta.reshape(1, KK)

    out = pl.pallas_call(
        _apply_body,
        grid=(BB, pl.cdiv(NN, _TE)),
        in_specs=[
            pl.BlockSpec((NBR, 1, _TE, 16), lambda b, i: (0, b, i, 0)),
            pl.BlockSpec((1, _TE, 16), lambda b, i: (b, i, 0)),
            pl.BlockSpec((16, KK), lambda b, i: (0, 0)),
            pl.BlockSpec((8, KK), lambda b, i: (0, 0)),
            pl.BlockSpec((8, KK), lambda b, i: (0, 0)),
            pl.BlockSpec((1, KK), lambda b, i: (0, 0)),
            pl.BlockSpec((1, KK), lambda b, i: (0, 0)),
        ],
        out_specs=pl.BlockSpec((1, KK, _TE), lambda b, i: (b, 0, i)),
        out_shape=jax.ShapeDtypeStruct((BB, KK, NN), jnp.float32),
    )(g4, xnp, dnp, sums, sqs, gm, bt)

    return out
